# baseline (speedup 1.0000x reference)
import functools

import jax
import jax.numpy as jnp
from jax import lax
from jax.experimental import pallas as pl
from jax.experimental.pallas import tpu as pltpu
from jax.experimental.pallas import tpu_sc as plsc

KS = 2
B_, H_, W_, C_ = 2, 256, 256, 96
N = B_ * H_ * W_ * C_
TOTAL = B_ * H_ * KS * W_ * KS * C_
SHIFT = 19
RNG = 1 << SHIFT
NB = TOTAL >> SHIFT
NC, NS, L = 2, 16, 16
HALF = N // NC
NSW = 32
NG = NC * NSW
SWLEN = HALF // NSW
SUB = SWLEN // NS
CHUNK = 2048
NCH_SW = SUB // CHUNK
CNT = NG * NB * NS * L
GBLK = NB * NS * L
GRP_MAX = ((SWLEN + NB * 512 + 32767) // 32768) * 32768
GPT = NG // NS
N_PAD = NG * GRP_MAX
BPC = NB // NC
SEG = RNG // NS
ZB = 16384
ZSP = GRP_MAX // NS
CH_B = 4096
NCHB = SUB // CH_B
FCH = 32768

_mesh = plsc.VectorSubcoreMesh(
    core_axis_name="c", subcore_axis_name="s", num_cores=NC, num_subcores=NS)


def _lane0(v):
    lane = lax.iota(jnp.int32, L)
    return jnp.sum(jnp.where(lane == 0, v, 0))


@functools.partial(
    pl.kernel,
    out_type=jax.ShapeDtypeStruct((CNT,), jnp.int32),
    mesh=_mesh,
    compiler_params=pltpu.CompilerParams(needs_layout_passes=False),
    scratch_types=[
        pltpu.VMEM((CH_B,), jnp.int32),
        pltpu.VMEM((CH_B,), jnp.int32),
        pltpu.VMEM((NB * L,), jnp.int32),
        pltpu.SemaphoreType.DMA,
        pltpu.SemaphoreType.DMA,
    ],
)
def _count_kernel(idx_hbm, cnt_hbm, win0, win1, hist, sem0, sem1):
    core = lax.axis_index("c")
    t = lax.axis_index("s")
    lane = lax.iota(jnp.int32, L)
    ones = jnp.ones((L,), jnp.int32)
    zeros = jnp.zeros((L,), jnp.int32)
    wins = (win0, win1)
    sems = (sem0, sem1)

    def _sw(sw, c0):
        g = core * NSW + sw
        base = core * HALF + sw * SWLEN + t * SUB

        def _z(b, c):
            hist[pl.ds(b * L, L)] = zeros
            return c

        lax.fori_loop(0, NB, _z, 0)

        ld = [None] * NCHB
        ld[0] = pltpu.async_copy(
            idx_hbm.at[pl.ds(base, CH_B)], wins[0], sems[0])
        for ci in range(NCHB):
            b = ci % 2
            ld[ci].wait()
            if ci + 1 < NCHB:
                nb = (ci + 1) % 2
                ld[ci + 1] = pltpu.async_copy(
                    idx_hbm.at[pl.ds(base + (ci + 1) * CH_B, CH_B)],
                    wins[nb], sems[nb])

            def _vec(j, cc):
                v = wins[b][pl.ds(j * L, L)]
                bk = jnp.right_shift(v, SHIFT)
                plsc.addupdate_scatter(hist, [bk * L + lane], ones)
                return cc

            lax.fori_loop(0, CH_B // L, _vec, 0)

        wd = []
        for b in range(NB):
            wd.append(pltpu.async_copy(
                hist.at[pl.ds(b * L, L)],
                cnt_hbm.at[pl.ds(((g * NB + b) * NS + t) * L, L)], sem0))
        for d in wd:
            d.wait()
        return c0

    lax.fori_loop(0, NSW, _sw, 0)


@functools.partial(
    pl.kernel,
    out_type=jax.ShapeDtypeStruct((CNT + L,), jnp.int32),
    mesh=_mesh,
    compiler_params=pltpu.CompilerParams(needs_layout_passes=False),
    scratch_types=[
        pltpu.VMEM((GPT * GBLK,), jnp.int32),
        pltpu.VMEM((NS * L,), jnp.int32),
        pltpu.VMEM((L,), jnp.int32),
        pltpu.VMEM_SHARED((NS * L,), jnp.int32),
    ],
)
def _scan_kernel(cnt_hbm, offs_hbm, gbuf, gts, tail, sgt):
    core = lax.axis_index("c")
    t = lax.axis_index("s")

    @pl.when(core == 0)
    def _():
        pltpu.sync_copy(cnt_hbm.at[pl.ds(GPT * t * GBLK, GPT * GBLK)], gbuf)

        def _grp(q, carry):
            def _bkt(b, carry_b):
                def _step(k, carry_k):
                    i = q * GBLK + b * NS * L + k * L
                    x = gbuf[pl.ds(i, L)]
                    incl = plsc.cumsum(x)
                    gbuf[pl.ds(i, L)] = incl - x + carry_k
                    return carry_k + jnp.sum(x)

                ce = lax.fori_loop(0, NS, _step, carry_b)
                return jnp.bitwise_and(ce + 511, -512)

            ce = lax.fori_loop(0, NB, _bkt, carry)
            return jnp.bitwise_and(ce + 32767, -32768)

        pair_total = lax.fori_loop(0, GPT, _grp, jnp.int32(0))
        tail[pl.ds(0, L)] = jnp.full((L,), 1, jnp.int32) * pair_total
        pltpu.sync_copy(tail, sgt.at[pl.ds(t * L, L)])
        plsc.subcore_barrier()
        pltpu.sync_copy(sgt, gts)

        def _base(tp, acc2):
            v = _lane0(gts[pl.ds(tp * L, L)])
            return acc2 + jnp.where(tp < t, v, 0)

        base = lax.fori_loop(0, NS, _base, jnp.int32(0))

        def _add(i, c):
            gbuf[pl.ds(i * L, L)] = gbuf[pl.ds(i * L, L)] + base
            return c

        lax.fori_loop(0, GPT * GBLK // L, _add, 0)
        pltpu.sync_copy(gbuf, offs_hbm.at[pl.ds(GPT * t * GBLK, GPT * GBLK)])

        @pl.when(t == NS - 1)
        def _tail():
            tail[pl.ds(0, L)] = jnp.full((L,), 1, jnp.int32) * (
                base + pair_total)
            pltpu.sync_copy(tail, offs_hbm.at[pl.ds(CNT, L)])


@functools.partial(
    pl.kernel,
    out_type=[
        jax.ShapeDtypeStruct((N_PAD,), jnp.int32),
        jax.ShapeDtypeStruct((N_PAD,), jnp.float32),
    ],
    mesh=_mesh,
    compiler_params=pltpu.CompilerParams(needs_layout_passes=False),
    scratch_types=[
        pltpu.VMEM((CH_B,), jnp.int32),
        pltpu.VMEM((CH_B,), jnp.int32),
        pltpu.VMEM((CH_B,), jnp.float32),
        pltpu.VMEM((CH_B,), jnp.float32),
        pltpu.VMEM((CH_B,), jnp.int32),
        pltpu.VMEM((CH_B,), jnp.int32),
        pltpu.VMEM((NB * L,), jnp.int32),
        pltpu.VMEM((L,), jnp.int32),
        pltpu.VMEM((ZSP,), jnp.float32),
        pltpu.VMEM((ZSP,), jnp.int32),
        pltpu.VMEM_SHARED((GRP_MAX,), jnp.int32),
        pltpu.VMEM_SHARED((GRP_MAX,), jnp.float32),
        pltpu.SemaphoreType.DMA,
        pltpu.SemaphoreType.DMA,
        pltpu.SemaphoreType.DMA,
        pltpu.SemaphoreType.DMA,
        pltpu.SemaphoreType.DMA,
        pltpu.SemaphoreType.DMA,
    ],
)
def _bin_kernel(idx_hbm, val_hbm, offs_hbm, bidx_hbm, bval_hbm,
                wi0, wi1, wv0, wv1, de0, de1, own, g16, zbuf, zbuf_i,
                sp_i, sp_v, sli0, sli1, slv0, slv1, ssc0, ssc1):
    core = lax.axis_index("c")
    t = lax.axis_index("s")
    lane = lax.iota(jnp.int32, L)
    fzeros = jnp.zeros((L,), jnp.float32)
    wis = (wi0, wi1)
    wvs = (wv0, wv1)
    des = (de0, de1)
    slis = (sli0, sli1)
    slvs = (slv0, slv1)
    sscs = (ssc0, ssc1)

    izeros = jnp.zeros((L,), jnp.int32)

    def _zz(i, c):
        zbuf[pl.ds(i * L, L)] = fzeros
        zbuf_i[pl.ds(i * L, L)] = izeros
        return c

    lax.fori_loop(0, ZSP // L, _zz, 0)

    def _sw(sw, c0):
        g = core * NSW + sw
        base = core * HALF + sw * SWLEN + t * SUB
        pltpu.sync_copy(offs_hbm.at[pl.ds(g * GBLK, L)], g16)
        gb = pl.multiple_of(_lane0(g16[pl.ds(0, L)]), 2048)
        pltpu.sync_copy(offs_hbm.at[pl.ds((g + 1) * GBLK, L)], g16)
        gn = pl.multiple_of(_lane0(g16[pl.ds(0, L)]), 2048)

        zd = pltpu.async_copy(zbuf, sp_v.at[pl.ds(t * ZSP, ZSP)], ssc0)
        zdi = pltpu.async_copy(zbuf_i, sp_i.at[pl.ds(t * ZSP, ZSP)], ssc1)

        cd = []
        for b in range(NB):
            cd.append(pltpu.async_copy(
                offs_hbm.at[pl.ds(((g * NB + b) * NS + t) * L, L)],
                own.at[pl.ds(b * L, L)], sli0))
        for d in cd:
            d.wait()
        zd.wait()
        zdi.wait()
        plsc.subcore_barrier()

        ld_i = [None] * NCHB
        ld_v = [None] * NCHB
        sc_i = [None] * NCHB
        sc_v = [None] * NCHB
        ld_i[0] = pltpu.async_copy(
            idx_hbm.at[pl.ds(base, CH_B)], wis[0], slis[0])
        ld_v[0] = pltpu.async_copy(
            val_hbm.at[pl.ds(base, CH_B)], wvs[0], slvs[0])
        for ci in range(NCHB):
            b = ci % 2
            ld_i[ci].wait()
            ld_v[ci].wait()
            if ci + 1 < NCHB:
                nb = (ci + 1) % 2
                if ci >= 1:
                    sc_i[ci - 1].wait()
                    sc_v[ci - 1].wait()
                ld_i[ci + 1] = pltpu.async_copy(
                    idx_hbm.at[pl.ds(base + (ci + 1) * CH_B, CH_B)],
                    wis[nb], slis[nb])
                ld_v[ci + 1] = pltpu.async_copy(
                    val_hbm.at[pl.ds(base + (ci + 1) * CH_B, CH_B)],
                    wvs[nb], slvs[nb])

            def _vec(j, cc):
                v = wis[b][pl.ds(j * L, L)]
                addr = jnp.right_shift(v, SHIFT) * L + lane
                cur = plsc.load_gather(own, [addr])
                plsc.store_scatter(own, [addr], cur + 1)
                wis[b][pl.ds(j * L, L)] = jnp.bitwise_and(v, RNG - 1)
                des[b][pl.ds(j * L, L)] = cur - gb
                return cc

            lax.fori_loop(0, CH_B // L, _vec, 0)
            sc_i[ci] = pltpu.async_copy(wis[b], sp_i.at[des[b]], sscs[b])
            sc_v[ci] = pltpu.async_copy(wvs[b], sp_v.at[des[b]], sscs[b])
        for ci in (NCHB - 2, NCHB - 1):
            sc_i[ci].wait()
            sc_v[ci].wait()
        plsc.subcore_barrier()

        nch = (gn - gb) // FCH

        def _fcond(ch):
            return ch < nch

        def _fbody(ch):
            pltpu.sync_copy(sp_i.at[pl.ds(ch * FCH, FCH)],
                            bidx_hbm.at[pl.ds(gb + ch * FCH, FCH)])
            pltpu.sync_copy(sp_v.at[pl.ds(ch * FCH, FCH)],
                            bval_hbm.at[pl.ds(gb + ch * FCH, FCH)])
            return ch + NS

        lax.while_loop(_fcond, _fbody, t)
        plsc.subcore_barrier()
        return c0

    lax.fori_loop(0, NSW, _sw, 0)


@functools.partial(
    pl.kernel,
    out_type=jax.ShapeDtypeStruct((TOTAL,), jnp.float32),
    mesh=_mesh,
    compiler_params=pltpu.CompilerParams(needs_layout_passes=False),
    scratch_types=[
        pltpu.VMEM((CHUNK,), jnp.int32),
        pltpu.VMEM((CHUNK,), jnp.float32),
        pltpu.VMEM((CHUNK,), jnp.int32),
        pltpu.VMEM((CHUNK,), jnp.float32),
        pltpu.VMEM((1024,), jnp.int32),
        pltpu.VMEM((1024,), jnp.float32),
        pltpu.VMEM((512,), jnp.int32),
        pltpu.VMEM((512,), jnp.float32),
        pltpu.VMEM((SEG,), jnp.float32),
        pltpu.VMEM((GPT * (NB + 1) * L,), jnp.int32),
        pltpu.VMEM_SHARED((RNG,), jnp.float32),
        pltpu.SemaphoreType.DMA,
        pltpu.SemaphoreType.DMA,
        pltpu.SemaphoreType.DMA,
        pltpu.SemaphoreType.DMA,
    ],
)
def _accum_kernel(bidx_hbm, bval_hbm, offs_hbm, out_hbm,
                  wia, wva, wib, wvb, w1i, w1v, w0i, w0v, zbuf, bnd, acc,
                  sla, slb, sca, scb):
    core = lax.axis_index("c")
    sid = lax.axis_index("s")
    fzeros = jnp.zeros((L,), jnp.float32)

    def _zz(i, c):
        zbuf[pl.ds(i * L, L)] = fzeros
        return c

    lax.fori_loop(0, SEG // L, _zz, 0)

    def _bn(b, c):
        for qq in range(GPT):
            pltpu.sync_copy(
                offs_hbm.at[pl.ds(((sid + qq * NS) * NB + b) * NS * L, L)],
                bnd.at[pl.ds((qq * (NB + 1) + b) * L, L)])
        return c

    lax.fori_loop(0, NB + 1, _bn, 0)

    def _bucket(jb, c):
        b = core * BPC + jb

        pltpu.sync_copy(zbuf, acc.at[pl.ds(sid * SEG, SEG)])
        plsc.subcore_barrier()

        for q in range(GPT):
            ss = pl.multiple_of(
                _lane0(bnd[pl.ds((q * (NB + 1) + b) * L, L)]), 512)
            re = pl.multiple_of(
                _lane0(bnd[pl.ds((q * (NB + 1) + b + 1) * L, L)]), 512)
            n2 = (re - ss) // (2 * CHUNK)

            def _pair(pi):
                el = ss + pi * 2 * CHUNK
                la = [pltpu.async_copy(
                    bidx_hbm.at[pl.ds(el, CHUNK)], wia, sla)]
                la.append(pltpu.async_copy(
                    bval_hbm.at[pl.ds(el, CHUNK)], wva, sla))
                lb = [pltpu.async_copy(
                    bidx_hbm.at[pl.ds(el + CHUNK, CHUNK)], wib, slb)]
                lb.append(pltpu.async_copy(
                    bval_hbm.at[pl.ds(el + CHUNK, CHUNK)], wvb, slb))
                for d in la:
                    d.wait()
                da = pltpu.async_copy(wva, acc.at[wia], sca, add=True)
                for d in lb:
                    d.wait()
                db = pltpu.async_copy(wvb, acc.at[wib], scb, add=True)
                da.wait()
                db.wait()
                return pi + 1

            lax.while_loop(lambda pi: pi < n2, _pair, jnp.int32(0))

            rem = re - (ss + n2 * 2 * CHUNK)
            p1 = ss + n2 * 2 * CHUNK

            @pl.when(rem >= 2048)
            def _t2():
                pltpu.sync_copy(bidx_hbm.at[pl.ds(p1, CHUNK)], wia)
                pltpu.sync_copy(bval_hbm.at[pl.ds(p1, CHUNK)], wva)
                pltpu.sync_copy(wva, acc.at[wia], add=True)

            p2 = p1 + jnp.where(rem >= 2048, 2048, 0)
            rem2 = rem - jnp.where(rem >= 2048, 2048, 0)

            @pl.when(rem2 >= 1024)
            def _t1():
                pltpu.sync_copy(bidx_hbm.at[pl.ds(p2, 1024)], w1i)
                pltpu.sync_copy(bval_hbm.at[pl.ds(p2, 1024)], w1v)
                pltpu.sync_copy(w1v, acc.at[w1i], add=True)

            p3 = p2 + jnp.where(rem2 >= 1024, 1024, 0)

            @pl.when((rem2 == 512) | (rem2 == 1536))
            def _t0():
                pltpu.sync_copy(bidx_hbm.at[pl.ds(p3, 512)], w0i)
                pltpu.sync_copy(bval_hbm.at[pl.ds(p3, 512)], w0v)
                pltpu.sync_copy(w0v, acc.at[w0i], add=True)

        plsc.subcore_barrier()

        pltpu.sync_copy(acc.at[pl.ds(sid * SEG, SEG)],
                        out_hbm.at[pl.ds(lo_out(b) + sid * SEG, SEG)])
        plsc.subcore_barrier()
        return c

    lax.fori_loop(0, BPC, _bucket, 0)


def lo_out(b):
    return b * RNG


def kernel(input, ind):
    vals = input.reshape(-1)
    idx = ind.reshape(-1).astype(jnp.int32)
    cnt = _count_kernel(idx)
    offs = _scan_kernel(cnt)
    bidx, bval = _bin_kernel(idx, vals, offs)
    out = _accum_kernel(bidx, bval, offs)
    return out.reshape(B_, H_ * KS, W_ * KS, C_)


# speedup vs baseline: 1.2444x; 1.2444x over previous
import functools

import jax
import jax.numpy as jnp
from jax import lax
from jax.experimental import pallas as pl
from jax.experimental.pallas import tpu as pltpu
from jax.experimental.pallas import tpu_sc as plsc

KS = 2
B_, H_, W_, C_ = 2, 256, 256, 96
N = B_ * H_ * W_ * C_
TOTAL = B_ * H_ * KS * W_ * KS * C_
SHIFT = 19
RNG = 1 << SHIFT
NB = TOTAL >> SHIFT
NC, NS, L = 2, 16, 16
HALF = N // NC
NSW = 16
NG = NC * NSW
SWLEN = HALF // NSW
SUB = SWLEN // NS
CHUNK = 2048
NCH_SW = SUB // CHUNK
CNT = NG * NB * NS * L
GBLK = NB * NS * L
GRP_MAX = ((SWLEN + NB * 64 + 8191) // 8192) * 8192
GPT = NG // NS
N_PAD = NG * GRP_MAX
BPC = NB // NC
SEG = RNG // NS
ZB = 16384
ZSP = GRP_MAX // NS
CH_B = 4096
NCHB = SUB // CH_B
FCH = 8192

_mesh = plsc.VectorSubcoreMesh(
    core_axis_name="c", subcore_axis_name="s", num_cores=NC, num_subcores=NS)


def _lane0(v):
    lane = lax.iota(jnp.int32, L)
    return jnp.sum(jnp.where(lane == 0, v, 0))


@functools.partial(
    pl.kernel,
    out_type=jax.ShapeDtypeStruct((CNT,), jnp.int32),
    mesh=_mesh,
    compiler_params=pltpu.CompilerParams(needs_layout_passes=False),
    scratch_types=[
        pltpu.VMEM((CH_B,), jnp.int32),
        pltpu.VMEM((CH_B,), jnp.int32),
        pltpu.VMEM((NB * L,), jnp.int32),
        pltpu.SemaphoreType.DMA,
        pltpu.SemaphoreType.DMA,
    ],
)
def _count_kernel(idx_hbm, cnt_hbm, win0, win1, hist, sem0, sem1):
    core = lax.axis_index("c")
    t = lax.axis_index("s")
    lane = lax.iota(jnp.int32, L)
    ones = jnp.ones((L,), jnp.int32)
    zeros = jnp.zeros((L,), jnp.int32)
    wins = (win0, win1)
    sems = (sem0, sem1)

    def _sw(sw, c0):
        g = core * NSW + sw
        base = core * HALF + sw * SWLEN + t * SUB

        def _z(b, c):
            hist[pl.ds(b * L, L)] = zeros
            return c

        lax.fori_loop(0, NB, _z, 0)

        ld = [None] * NCHB
        ld[0] = pltpu.async_copy(
            idx_hbm.at[pl.ds(base, CH_B)], wins[0], sems[0])
        for ci in range(NCHB):
            b = ci % 2
            ld[ci].wait()
            if ci + 1 < NCHB:
                nb = (ci + 1) % 2
                ld[ci + 1] = pltpu.async_copy(
                    idx_hbm.at[pl.ds(base + (ci + 1) * CH_B, CH_B)],
                    wins[nb], sems[nb])

            def _vec(j, cc):
                v = wins[b][pl.ds(j * L, L)]
                bk = jnp.right_shift(v, SHIFT)
                plsc.addupdate_scatter(hist, [bk * L + lane], ones)
                return cc

            lax.fori_loop(0, CH_B // L, _vec, 0)

        wd = []
        for b in range(NB):
            wd.append(pltpu.async_copy(
                hist.at[pl.ds(b * L, L)],
                cnt_hbm.at[pl.ds(((g * NB + b) * NS + t) * L, L)], sem0))
        for d in wd:
            d.wait()
        return c0

    lax.fori_loop(0, NSW, _sw, 0)


@functools.partial(
    pl.kernel,
    out_type=jax.ShapeDtypeStruct((CNT + L,), jnp.int32),
    mesh=_mesh,
    compiler_params=pltpu.CompilerParams(needs_layout_passes=False),
    scratch_types=[
        pltpu.VMEM((GPT * GBLK,), jnp.int32),
        pltpu.VMEM((NS * L,), jnp.int32),
        pltpu.VMEM((L,), jnp.int32),
        pltpu.VMEM_SHARED((NS * L,), jnp.int32),
    ],
)
def _scan_kernel(cnt_hbm, offs_hbm, gbuf, gts, tail, sgt):
    core = lax.axis_index("c")
    t = lax.axis_index("s")

    @pl.when(core == 0)
    def _():
        pltpu.sync_copy(cnt_hbm.at[pl.ds(GPT * t * GBLK, GPT * GBLK)], gbuf)

        def _grp(q, carry):
            def _bkt(b, carry_b):
                def _step(k, carry_k):
                    i = q * GBLK + b * NS * L + k * L
                    x = gbuf[pl.ds(i, L)]
                    incl = plsc.cumsum(x)
                    gbuf[pl.ds(i, L)] = incl - x + carry_k
                    return carry_k + jnp.sum(x)

                ce = lax.fori_loop(0, NS, _step, carry_b)
                return jnp.bitwise_and(ce + 63, -64)

            ce = lax.fori_loop(0, NB, _bkt, carry)
            return jnp.bitwise_and(ce + 8191, -8192)

        pair_total = lax.fori_loop(0, GPT, _grp, jnp.int32(0))
        tail[pl.ds(0, L)] = jnp.full((L,), 1, jnp.int32) * pair_total
        pltpu.sync_copy(tail, sgt.at[pl.ds(t * L, L)])
        plsc.subcore_barrier()
        pltpu.sync_copy(sgt, gts)

        def _base(tp, acc2):
            v = _lane0(gts[pl.ds(tp * L, L)])
            return acc2 + jnp.where(tp < t, v, 0)

        base = lax.fori_loop(0, NS, _base, jnp.int32(0))

        def _add(i, c):
            gbuf[pl.ds(i * L, L)] = gbuf[pl.ds(i * L, L)] + base
            return c

        lax.fori_loop(0, GPT * GBLK // L, _add, 0)
        pltpu.sync_copy(gbuf, offs_hbm.at[pl.ds(GPT * t * GBLK, GPT * GBLK)])

        @pl.when(t == NS - 1)
        def _tail():
            tail[pl.ds(0, L)] = jnp.full((L,), 1, jnp.int32) * (
                base + pair_total)
            pltpu.sync_copy(tail, offs_hbm.at[pl.ds(CNT, L)])


@functools.partial(
    pl.kernel,
    out_type=[
        jax.ShapeDtypeStruct((N_PAD,), jnp.int32),
        jax.ShapeDtypeStruct((N_PAD,), jnp.float32),
    ],
    mesh=_mesh,
    compiler_params=pltpu.CompilerParams(needs_layout_passes=False),
    scratch_types=[
        pltpu.VMEM((CH_B,), jnp.int32),
        pltpu.VMEM((CH_B,), jnp.int32),
        pltpu.VMEM((CH_B,), jnp.float32),
        pltpu.VMEM((CH_B,), jnp.float32),
        pltpu.VMEM((CH_B,), jnp.int32),
        pltpu.VMEM((CH_B,), jnp.int32),
        pltpu.VMEM((NB * L,), jnp.int32),
        pltpu.VMEM((L,), jnp.int32),
        pltpu.VMEM((ZSP,), jnp.float32),
        pltpu.VMEM((ZSP,), jnp.int32),
        pltpu.VMEM_SHARED((GRP_MAX,), jnp.int32),
        pltpu.VMEM_SHARED((GRP_MAX,), jnp.float32),
        pltpu.SemaphoreType.DMA,
        pltpu.SemaphoreType.DMA,
        pltpu.SemaphoreType.DMA,
        pltpu.SemaphoreType.DMA,
        pltpu.SemaphoreType.DMA,
        pltpu.SemaphoreType.DMA,
    ],
)
def _bin_kernel(idx_hbm, val_hbm, offs_hbm, bidx_hbm, bval_hbm,
                wi0, wi1, wv0, wv1, de0, de1, own, g16, zbuf, zbuf_i,
                sp_i, sp_v, sli0, sli1, slv0, slv1, ssc0, ssc1):
    core = lax.axis_index("c")
    t = lax.axis_index("s")
    lane = lax.iota(jnp.int32, L)
    fzeros = jnp.zeros((L,), jnp.float32)
    wis = (wi0, wi1)
    wvs = (wv0, wv1)
    des = (de0, de1)
    slis = (sli0, sli1)
    slvs = (slv0, slv1)
    sscs = (ssc0, ssc1)

    izeros = jnp.zeros((L,), jnp.int32)

    def _zz(i, c):
        zbuf[pl.ds(i * L, L)] = fzeros
        zbuf_i[pl.ds(i * L, L)] = izeros
        return c

    lax.fori_loop(0, ZSP // L, _zz, 0)

    def _sw(sw, c0):
        g = core * NSW + sw
        base = core * HALF + sw * SWLEN + t * SUB
        pltpu.sync_copy(offs_hbm.at[pl.ds(g * GBLK, L)], g16)
        gb = pl.multiple_of(_lane0(g16[pl.ds(0, L)]), 2048)
        pltpu.sync_copy(offs_hbm.at[pl.ds((g + 1) * GBLK, L)], g16)
        gn = pl.multiple_of(_lane0(g16[pl.ds(0, L)]), 2048)

        zd = pltpu.async_copy(zbuf, sp_v.at[pl.ds(t * ZSP, ZSP)], ssc0)
        zdi = pltpu.async_copy(zbuf_i, sp_i.at[pl.ds(t * ZSP, ZSP)], ssc1)

        cd = []
        for b in range(NB):
            cd.append(pltpu.async_copy(
                offs_hbm.at[pl.ds(((g * NB + b) * NS + t) * L, L)],
                own.at[pl.ds(b * L, L)], sli0))
        for d in cd:
            d.wait()
        zd.wait()
        zdi.wait()
        plsc.subcore_barrier()

        ld_i = [None] * NCHB
        ld_v = [None] * NCHB
        sc_i = [None] * NCHB
        sc_v = [None] * NCHB
        ld_i[0] = pltpu.async_copy(
            idx_hbm.at[pl.ds(base, CH_B)], wis[0], slis[0])
        ld_v[0] = pltpu.async_copy(
            val_hbm.at[pl.ds(base, CH_B)], wvs[0], slvs[0])
        for ci in range(NCHB):
            b = ci % 2
            ld_i[ci].wait()
            ld_v[ci].wait()
            if ci + 1 < NCHB:
                nb = (ci + 1) % 2
                if ci >= 1:
                    sc_i[ci - 1].wait()
                    sc_v[ci - 1].wait()
                ld_i[ci + 1] = pltpu.async_copy(
                    idx_hbm.at[pl.ds(base + (ci + 1) * CH_B, CH_B)],
                    wis[nb], slis[nb])
                ld_v[ci + 1] = pltpu.async_copy(
                    val_hbm.at[pl.ds(base + (ci + 1) * CH_B, CH_B)],
                    wvs[nb], slvs[nb])

            def _vec(j, cc):
                v = wis[b][pl.ds(j * L, L)]
                addr = jnp.right_shift(v, SHIFT) * L + lane
                cur = plsc.load_gather(own, [addr])
                plsc.store_scatter(own, [addr], cur + 1)
                des[b][pl.ds(j * L, L)] = cur - gb
                return cc

            lax.fori_loop(0, CH_B // L, _vec, 0)
            sc_i[ci] = pltpu.async_copy(wis[b], sp_i.at[des[b]], sscs[b])
            sc_v[ci] = pltpu.async_copy(wvs[b], sp_v.at[des[b]], sscs[b])
        for ci in (NCHB - 2, NCHB - 1):
            sc_i[ci].wait()
            sc_v[ci].wait()
        plsc.subcore_barrier()

        nch = (gn - gb) // FCH

        def _fcond(ch):
            return ch < nch

        def _fbody(ch):
            pltpu.sync_copy(sp_i.at[pl.ds(ch * FCH, FCH)],
                            bidx_hbm.at[pl.ds(gb + ch * FCH, FCH)])
            pltpu.sync_copy(sp_v.at[pl.ds(ch * FCH, FCH)],
                            bval_hbm.at[pl.ds(gb + ch * FCH, FCH)])
            return ch + NS

        lax.while_loop(_fcond, _fbody, t)
        plsc.subcore_barrier()
        return c0

    lax.fori_loop(0, NSW, _sw, 0)


@functools.partial(
    pl.kernel,
    out_type=jax.ShapeDtypeStruct((TOTAL,), jnp.float32),
    mesh=_mesh,
    compiler_params=pltpu.CompilerParams(needs_layout_passes=False),
    scratch_types=[
        pltpu.VMEM((CHUNK,), jnp.int32),
        pltpu.VMEM((CHUNK,), jnp.float32),
        pltpu.VMEM((CHUNK,), jnp.int32),
        pltpu.VMEM((CHUNK,), jnp.float32),
        pltpu.VMEM((CHUNK,), jnp.int32),
        pltpu.VMEM((CHUNK,), jnp.float32),
        pltpu.VMEM((CHUNK,), jnp.int32),
        pltpu.VMEM((CHUNK,), jnp.float32),
        pltpu.VMEM((SEG,), jnp.float32),
        pltpu.VMEM((GPT * (NB + 1) * L,), jnp.int32),
        pltpu.VMEM_SHARED((RNG,), jnp.float32),
        pltpu.SemaphoreType.DMA,
        pltpu.SemaphoreType.DMA,
        pltpu.SemaphoreType.DMA,
        pltpu.SemaphoreType.DMA,
    ],
)
def _accum_kernel(bidx_hbm, bval_hbm, offs_hbm, out_hbm,
                  wia, wva, wib, wvb, w1i, w1v, w0i, w0v, zbuf, bnd, acc,
                  sla, slb, sca, scb):
    core = lax.axis_index("c")
    sid = lax.axis_index("s")
    fzeros = jnp.zeros((L,), jnp.float32)

    def _zz(i, c):
        zbuf[pl.ds(i * L, L)] = fzeros
        return c

    lax.fori_loop(0, SEG // L, _zz, 0)

    def _bn(b, c):
        for qq in range(GPT):
            pltpu.sync_copy(
                offs_hbm.at[pl.ds(((sid + qq * NS) * NB + b) * NS * L, L)],
                bnd.at[pl.ds((qq * (NB + 1) + b) * L, L)])
        return c

    lax.fori_loop(0, NB + 1, _bn, 0)

    def _bucket(jb, c):
        b = core * BPC + jb

        pltpu.sync_copy(zbuf, acc.at[pl.ds(sid * SEG, SEG)])
        plsc.subcore_barrier()

        lo = b * RNG

        for q in range(GPT):
            ss = _lane0(bnd[pl.ds((q * (NB + 1) + b) * L, L)])
            re = _lane0(bnd[pl.ds((q * (NB + 1) + b + 1) * L, L)])
            c0 = ss // CHUNK
            c1 = (re + CHUNK - 1) // CHUNK

            def _body(ci):
                pltpu.sync_copy(bidx_hbm.at[ci], wia)
                pltpu.sync_copy(bval_hbm.at[ci], wva)

                def _vec(j, cc):
                    lv = wia[pl.ds(j * L, L)] - lo
                    m = (lv >= 0) & (lv < RNG)
                    w1i[pl.ds(j * L, L)] = jnp.bitwise_and(lv, RNG - 1)
                    w1v[pl.ds(j * L, L)] = jnp.where(
                        m, wva[pl.ds(j * L, L)], 0.0)
                    return cc

                lax.fori_loop(0, CHUNK // L, _vec, 0)
                pltpu.sync_copy(w1v, acc.at[w1i], add=True)
                return ci + 1

            lax.while_loop(lambda ci: ci < c1, _body, c0)

        plsc.subcore_barrier()

        pltpu.sync_copy(acc.at[pl.ds(sid * SEG, SEG)],
                        out_hbm.at[pl.ds(lo + sid * SEG, SEG)])
        plsc.subcore_barrier()
        return c

    lax.fori_loop(0, BPC, _bucket, 0)


def lo_out(b):
    return b * RNG


def kernel(input, ind):
    vals = input.reshape(-1)
    idx = ind.reshape(-1).astype(jnp.int32)
    cnt = _count_kernel(idx)
    offs = _scan_kernel(cnt)
    bidx, bval = _bin_kernel(idx, vals, offs)
    out = _accum_kernel(bidx.reshape(-1, CHUNK), bval.reshape(-1, CHUNK), offs)
    return out.reshape(B_, H_ * KS, W_ * KS, C_)


# speedup vs baseline: 1.3015x; 1.0459x over previous
import functools

import jax
import jax.numpy as jnp
from jax import lax
from jax.experimental import pallas as pl
from jax.experimental.pallas import tpu as pltpu
from jax.experimental.pallas import tpu_sc as plsc

KS = 2
B_, H_, W_, C_ = 2, 256, 256, 96
N = B_ * H_ * W_ * C_
TOTAL = B_ * H_ * KS * W_ * KS * C_
SHIFT = 19
RNG = 1 << SHIFT
NB = TOTAL >> SHIFT
NC, NS, L = 2, 16, 16
HALF = N // NC
NSW = 16
NG = NC * NSW
SWLEN = HALF // NSW
SUB = SWLEN // NS
CHUNK = 2048
NCH_SW = SUB // CHUNK
CNT = NG * NB * NS * L
GBLK = NB * NS * L
GRP_MAX = ((SWLEN + NB * 64 + 8191) // 8192) * 8192
GPT = NG // NS
N_PAD = NG * GRP_MAX
BPC = NB // NC
SEG = RNG // NS
ZB = 16384
ZSP = GRP_MAX // NS
CH_B = 4096
NCHB = SUB // CH_B
FCH = 8192

_mesh = plsc.VectorSubcoreMesh(
    core_axis_name="c", subcore_axis_name="s", num_cores=NC, num_subcores=NS)


def _lane0(v):
    lane = lax.iota(jnp.int32, L)
    return jnp.sum(jnp.where(lane == 0, v, 0))


@functools.partial(
    pl.kernel,
    out_type=jax.ShapeDtypeStruct((CNT,), jnp.int32),
    mesh=_mesh,
    compiler_params=pltpu.CompilerParams(needs_layout_passes=False),
    scratch_types=[
        pltpu.VMEM((CH_B,), jnp.int32),
        pltpu.VMEM((CH_B,), jnp.int32),
        pltpu.VMEM((NB * L,), jnp.int32),
        pltpu.SemaphoreType.DMA,
        pltpu.SemaphoreType.DMA,
    ],
)
def _count_kernel(idx_hbm, cnt_hbm, win0, win1, hist, sem0, sem1):
    core = lax.axis_index("c")
    t = lax.axis_index("s")
    lane = lax.iota(jnp.int32, L)
    ones = jnp.ones((L,), jnp.int32)
    zeros = jnp.zeros((L,), jnp.int32)
    wins = (win0, win1)
    sems = (sem0, sem1)

    def _sw(sw, c0):
        g = core * NSW + sw
        base = core * HALF + sw * SWLEN + t * SUB

        def _z(b, c):
            hist[pl.ds(b * L, L)] = zeros
            return c

        lax.fori_loop(0, NB, _z, 0)

        ld = [None] * NCHB
        ld[0] = pltpu.async_copy(
            idx_hbm.at[pl.ds(base, CH_B)], wins[0], sems[0])
        for ci in range(NCHB):
            b = ci % 2
            ld[ci].wait()
            if ci + 1 < NCHB:
                nb = (ci + 1) % 2
                ld[ci + 1] = pltpu.async_copy(
                    idx_hbm.at[pl.ds(base + (ci + 1) * CH_B, CH_B)],
                    wins[nb], sems[nb])

            def _vec(j, cc):
                v = wins[b][pl.ds(j * L, L)]
                bk = jnp.right_shift(v, SHIFT)
                plsc.addupdate_scatter(hist, [bk * L + lane], ones)
                return cc

            lax.fori_loop(0, CH_B // L, _vec, 0)

        wd = []
        for b in range(NB):
            wd.append(pltpu.async_copy(
                hist.at[pl.ds(b * L, L)],
                cnt_hbm.at[pl.ds(((g * NB + b) * NS + t) * L, L)], sem0))
        for d in wd:
            d.wait()
        return c0

    lax.fori_loop(0, NSW, _sw, 0)


@functools.partial(
    pl.kernel,
    out_type=jax.ShapeDtypeStruct((CNT + L,), jnp.int32),
    mesh=_mesh,
    compiler_params=pltpu.CompilerParams(needs_layout_passes=False),
    scratch_types=[
        pltpu.VMEM((GPT * GBLK,), jnp.int32),
        pltpu.VMEM((NS * L,), jnp.int32),
        pltpu.VMEM((L,), jnp.int32),
        pltpu.VMEM_SHARED((NS * L,), jnp.int32),
    ],
)
def _scan_kernel(cnt_hbm, offs_hbm, gbuf, gts, tail, sgt):
    core = lax.axis_index("c")
    t = lax.axis_index("s")

    @pl.when(core == 0)
    def _():
        pltpu.sync_copy(cnt_hbm.at[pl.ds(GPT * t * GBLK, GPT * GBLK)], gbuf)

        def _grp(q, carry):
            def _bkt(b, carry_b):
                def _step(k, carry_k):
                    i = q * GBLK + b * NS * L + k * L
                    x = gbuf[pl.ds(i, L)]
                    incl = plsc.cumsum(x)
                    gbuf[pl.ds(i, L)] = incl - x + carry_k
                    return carry_k + jnp.sum(x)

                ce = lax.fori_loop(0, NS, _step, carry_b)
                return jnp.bitwise_and(ce + 63, -64)

            ce = lax.fori_loop(0, NB, _bkt, carry)
            return jnp.bitwise_and(ce + 8191, -8192)

        pair_total = lax.fori_loop(0, GPT, _grp, jnp.int32(0))
        tail[pl.ds(0, L)] = jnp.full((L,), 1, jnp.int32) * pair_total
        pltpu.sync_copy(tail, sgt.at[pl.ds(t * L, L)])
        plsc.subcore_barrier()
        pltpu.sync_copy(sgt, gts)

        def _base(tp, acc2):
            v = _lane0(gts[pl.ds(tp * L, L)])
            return acc2 + jnp.where(tp < t, v, 0)

        base = lax.fori_loop(0, NS, _base, jnp.int32(0))

        def _add(i, c):
            gbuf[pl.ds(i * L, L)] = gbuf[pl.ds(i * L, L)] + base
            return c

        lax.fori_loop(0, GPT * GBLK // L, _add, 0)
        pltpu.sync_copy(gbuf, offs_hbm.at[pl.ds(GPT * t * GBLK, GPT * GBLK)])

        @pl.when(t == NS - 1)
        def _tail():
            tail[pl.ds(0, L)] = jnp.full((L,), 1, jnp.int32) * (
                base + pair_total)
            pltpu.sync_copy(tail, offs_hbm.at[pl.ds(CNT, L)])


@functools.partial(
    pl.kernel,
    out_type=[
        jax.ShapeDtypeStruct((N_PAD,), jnp.int32),
        jax.ShapeDtypeStruct((N_PAD,), jnp.float32),
    ],
    mesh=_mesh,
    compiler_params=pltpu.CompilerParams(needs_layout_passes=False),
    scratch_types=[
        pltpu.VMEM((CH_B,), jnp.int32),
        pltpu.VMEM((CH_B,), jnp.int32),
        pltpu.VMEM((CH_B,), jnp.float32),
        pltpu.VMEM((CH_B,), jnp.float32),
        pltpu.VMEM((CH_B,), jnp.int32),
        pltpu.VMEM((CH_B,), jnp.int32),
        pltpu.VMEM((NB * L,), jnp.int32),
        pltpu.VMEM((L,), jnp.int32),
        pltpu.VMEM((ZSP,), jnp.float32),
        pltpu.VMEM((ZSP,), jnp.int32),
        pltpu.VMEM_SHARED((GRP_MAX,), jnp.int32),
        pltpu.VMEM_SHARED((GRP_MAX,), jnp.float32),
        pltpu.SemaphoreType.DMA,
        pltpu.SemaphoreType.DMA,
        pltpu.SemaphoreType.DMA,
        pltpu.SemaphoreType.DMA,
        pltpu.SemaphoreType.DMA,
        pltpu.SemaphoreType.DMA,
    ],
)
def _bin_kernel(idx_hbm, val_hbm, offs_hbm, bidx_hbm, bval_hbm,
                wi0, wi1, wv0, wv1, de0, de1, own, g16, zbuf, zbuf_i,
                sp_i, sp_v, sli0, sli1, slv0, slv1, ssc0, ssc1):
    core = lax.axis_index("c")
    t = lax.axis_index("s")
    lane = lax.iota(jnp.int32, L)
    fzeros = jnp.zeros((L,), jnp.float32)
    wis = (wi0, wi1)
    wvs = (wv0, wv1)
    des = (de0, de1)
    slis = (sli0, sli1)
    slvs = (slv0, slv1)
    sscs = (ssc0, ssc1)

    izeros = jnp.zeros((L,), jnp.int32)

    def _zz(i, c):
        zbuf[pl.ds(i * L, L)] = fzeros
        zbuf_i[pl.ds(i * L, L)] = izeros
        return c

    lax.fori_loop(0, ZSP // L, _zz, 0)

    def _sw(sw, c0):
        g = core * NSW + sw
        base = core * HALF + sw * SWLEN + t * SUB
        pltpu.sync_copy(offs_hbm.at[pl.ds(g * GBLK, L)], g16)
        gb = pl.multiple_of(_lane0(g16[pl.ds(0, L)]), 2048)
        pltpu.sync_copy(offs_hbm.at[pl.ds((g + 1) * GBLK, L)], g16)
        gn = pl.multiple_of(_lane0(g16[pl.ds(0, L)]), 2048)

        zd = pltpu.async_copy(zbuf, sp_v.at[pl.ds(t * ZSP, ZSP)], ssc0)
        zdi = pltpu.async_copy(zbuf_i, sp_i.at[pl.ds(t * ZSP, ZSP)], ssc1)

        cd = []
        for b in range(NB):
            cd.append(pltpu.async_copy(
                offs_hbm.at[pl.ds(((g * NB + b) * NS + t) * L, L)],
                own.at[pl.ds(b * L, L)], sli0))
        for d in cd:
            d.wait()
        zd.wait()
        zdi.wait()
        plsc.subcore_barrier()

        ld_i = [None] * NCHB
        ld_v = [None] * NCHB
        sc_i = [None] * NCHB
        sc_v = [None] * NCHB
        ld_i[0] = pltpu.async_copy(
            idx_hbm.at[pl.ds(base, CH_B)], wis[0], slis[0])
        ld_v[0] = pltpu.async_copy(
            val_hbm.at[pl.ds(base, CH_B)], wvs[0], slvs[0])
        for ci in range(NCHB):
            b = ci % 2
            ld_i[ci].wait()
            ld_v[ci].wait()
            if ci + 1 < NCHB:
                nb = (ci + 1) % 2
                if ci >= 1:
                    sc_i[ci - 1].wait()
                    sc_v[ci - 1].wait()
                ld_i[ci + 1] = pltpu.async_copy(
                    idx_hbm.at[pl.ds(base + (ci + 1) * CH_B, CH_B)],
                    wis[nb], slis[nb])
                ld_v[ci + 1] = pltpu.async_copy(
                    val_hbm.at[pl.ds(base + (ci + 1) * CH_B, CH_B)],
                    wvs[nb], slvs[nb])

            def _vec(j, cc):
                v = wis[b][pl.ds(j * L, L)]
                addr = jnp.right_shift(v, SHIFT) * L + lane
                cur = plsc.load_gather(own, [addr])
                plsc.store_scatter(own, [addr], cur + 1)
                des[b][pl.ds(j * L, L)] = cur - gb
                return cc

            lax.fori_loop(0, CH_B // L, _vec, 0)
            sc_i[ci] = pltpu.async_copy(wis[b], sp_i.at[des[b]], sscs[b])
            sc_v[ci] = pltpu.async_copy(wvs[b], sp_v.at[des[b]], sscs[b])
        for ci in (NCHB - 2, NCHB - 1):
            sc_i[ci].wait()
            sc_v[ci].wait()
        plsc.subcore_barrier()

        nch = (gn - gb) // FCH

        def _fcond(ch):
            return ch < nch

        def _fbody(ch):
            pltpu.sync_copy(sp_i.at[pl.ds(ch * FCH, FCH)],
                            bidx_hbm.at[pl.ds(gb + ch * FCH, FCH)])
            pltpu.sync_copy(sp_v.at[pl.ds(ch * FCH, FCH)],
                            bval_hbm.at[pl.ds(gb + ch * FCH, FCH)])
            return ch + NS

        lax.while_loop(_fcond, _fbody, t)
        plsc.subcore_barrier()
        return c0

    lax.fori_loop(0, NSW, _sw, 0)


@functools.partial(
    pl.kernel,
    out_type=jax.ShapeDtypeStruct((TOTAL,), jnp.float32),
    mesh=_mesh,
    compiler_params=pltpu.CompilerParams(needs_layout_passes=False),
    scratch_types=[
        pltpu.VMEM((CHUNK,), jnp.int32),
        pltpu.VMEM((CHUNK,), jnp.float32),
        pltpu.VMEM((CHUNK,), jnp.int32),
        pltpu.VMEM((CHUNK,), jnp.float32),
        pltpu.VMEM((CHUNK,), jnp.int32),
        pltpu.VMEM((CHUNK,), jnp.float32),
        pltpu.VMEM((CHUNK,), jnp.int32),
        pltpu.VMEM((CHUNK,), jnp.float32),
        pltpu.VMEM((SEG,), jnp.float32),
        pltpu.VMEM((GPT * (NB + 1) * L,), jnp.int32),
        pltpu.VMEM_SHARED((RNG,), jnp.float32),
        pltpu.SemaphoreType.DMA,
        pltpu.SemaphoreType.DMA,
        pltpu.SemaphoreType.DMA,
        pltpu.SemaphoreType.DMA,
    ],
)
def _accum_kernel(bidx_hbm, bval_hbm, offs_hbm, out_hbm,
                  wia, wva, wib, wvb, w1i, w1v, w0i, w0v, zbuf, bnd, acc,
                  sla, slb, sca, scb):
    core = lax.axis_index("c")
    sid = lax.axis_index("s")
    fzeros = jnp.zeros((L,), jnp.float32)

    def _zz(i, c):
        zbuf[pl.ds(i * L, L)] = fzeros
        return c

    lax.fori_loop(0, SEG // L, _zz, 0)

    def _bn(b, c):
        for qq in range(GPT):
            pltpu.sync_copy(
                offs_hbm.at[pl.ds(((sid + qq * NS) * NB + b) * NS * L, L)],
                bnd.at[pl.ds((qq * (NB + 1) + b) * L, L)])
        return c

    lax.fori_loop(0, NB + 1, _bn, 0)

    def _bucket(jb, c):
        b = core * BPC + jb

        pltpu.sync_copy(zbuf, acc.at[pl.ds(sid * SEG, SEG)])
        plsc.subcore_barrier()

        lo = b * RNG

        for q in range(GPT):
            ss = _lane0(bnd[pl.ds((q * (NB + 1) + b) * L, L)])
            re = _lane0(bnd[pl.ds((q * (NB + 1) + b + 1) * L, L)])
            c0 = ss // CHUNK
            c1 = (re + CHUNK - 1) // CHUNK

            def _vecf(wi, wv, di, dv, ok):
                def _vec(j, cc):
                    lv = wi[pl.ds(j * L, L)] - lo
                    m = (lv >= 0) & (lv < RNG) & ok
                    di[pl.ds(j * L, L)] = jnp.bitwise_and(lv, RNG - 1)
                    dv[pl.ds(j * L, L)] = jnp.where(
                        m, wv[pl.ds(j * L, L)], 0.0)
                    return cc

                lax.fori_loop(0, CHUNK // L, _vec, 0)

            def _body(ci):
                cb = jnp.minimum(ci + 1, c1 - 1)
                la_i = pltpu.async_copy(bidx_hbm.at[ci], wia, sla)
                la_v = pltpu.async_copy(bval_hbm.at[ci], wva, sla)
                lb_i = pltpu.async_copy(bidx_hbm.at[cb], wib, slb)
                lb_v = pltpu.async_copy(bval_hbm.at[cb], wvb, slb)
                la_i.wait()
                la_v.wait()
                _vecf(wia, wva, w1i, w1v, ci < c1)
                da = pltpu.async_copy(w1v, acc.at[w1i], sca, add=True)
                lb_i.wait()
                lb_v.wait()
                _vecf(wib, wvb, w0i, w0v, ci + 1 < c1)
                db = pltpu.async_copy(w0v, acc.at[w0i], scb, add=True)
                da.wait()
                db.wait()
                return ci + 2

            lax.while_loop(lambda ci: ci < c1, _body, c0)

        plsc.subcore_barrier()

        pltpu.sync_copy(acc.at[pl.ds(sid * SEG, SEG)],
                        out_hbm.at[pl.ds(lo + sid * SEG, SEG)])
        plsc.subcore_barrier()
        return c

    lax.fori_loop(0, BPC, _bucket, 0)


def lo_out(b):
    return b * RNG


def kernel(input, ind):
    vals = input.reshape(-1)
    idx = ind.reshape(-1).astype(jnp.int32)
    cnt = _count_kernel(idx)
    offs = _scan_kernel(cnt)
    bidx, bval = _bin_kernel(idx, vals, offs)
    out = _accum_kernel(bidx.reshape(-1, CHUNK), bval.reshape(-1, CHUNK), offs)
    return out.reshape(B_, H_ * KS, W_ * KS, C_)


# speedup vs baseline: 1.3020x; 1.0004x over previous
import functools

import jax
import jax.numpy as jnp
from jax import lax
from jax.experimental import pallas as pl
from jax.experimental.pallas import tpu as pltpu
from jax.experimental.pallas import tpu_sc as plsc

KS = 2
B_, H_, W_, C_ = 2, 256, 256, 96
N = B_ * H_ * W_ * C_
TOTAL = B_ * H_ * KS * W_ * KS * C_
SHIFT = 19
RNG = 1 << SHIFT
NB = TOTAL >> SHIFT
NC, NS, L = 2, 16, 16
HALF = N // NC
NSW = 16
NG = NC * NSW
SWLEN = HALF // NSW
SUB = SWLEN // NS
CHUNK = 2048
CNT = NG * NB * NS * L
GBLK = NB * NS * L
GRP_MAX = ((SWLEN + NB * 64 + 8191) // 8192) * 8192
GPT = NG // NS
N_PAD = NG * GRP_MAX
BPC = NB // NC
SEG = RNG // NS
ZSP = GRP_MAX // NS
CH_B = 4096
NCHB = SUB // CH_B
FCH = 8192

_mesh = plsc.VectorSubcoreMesh(
    core_axis_name="c", subcore_axis_name="s", num_cores=NC, num_subcores=NS)


def _lane0(v):
    lane = lax.iota(jnp.int32, L)
    return jnp.sum(jnp.where(lane == 0, v, 0))


@functools.partial(
    pl.kernel,
    out_type=jax.ShapeDtypeStruct((CNT,), jnp.int32),
    mesh=_mesh,
    compiler_params=pltpu.CompilerParams(needs_layout_passes=False),
    scratch_types=[
        pltpu.VMEM((CH_B,), jnp.int32),
        pltpu.VMEM((CH_B,), jnp.int32),
        pltpu.VMEM((NB * L,), jnp.int32),
        pltpu.SemaphoreType.DMA,
        pltpu.SemaphoreType.DMA,
    ],
)
def _count_kernel(idx_hbm, cnt_hbm, win0, win1, hist, sem0, sem1):
    core = lax.axis_index("c")
    t = lax.axis_index("s")
    lane = lax.iota(jnp.int32, L)
    ones = jnp.ones((L,), jnp.int32)
    zeros = jnp.zeros((L,), jnp.int32)
    wins = (win0, win1)
    sems = (sem0, sem1)

    def _sw(sw, c0):
        g = core * NSW + sw
        base = core * HALF + sw * SWLEN + t * SUB

        def _z(b, c):
            hist[pl.ds(b * L, L)] = zeros
            return c

        lax.fori_loop(0, NB, _z, 0)

        ld = [None] * NCHB
        ld[0] = pltpu.async_copy(
            idx_hbm.at[pl.ds(base, CH_B)], wins[0], sems[0])
        for ci in range(NCHB):
            b = ci % 2
            ld[ci].wait()
            if ci + 1 < NCHB:
                nb = (ci + 1) % 2
                ld[ci + 1] = pltpu.async_copy(
                    idx_hbm.at[pl.ds(base + (ci + 1) * CH_B, CH_B)],
                    wins[nb], sems[nb])

            def _vec(j, cc):
                v = wins[b][pl.ds(j * L, L)]
                bk = jnp.right_shift(v, SHIFT)
                plsc.addupdate_scatter(hist, [bk * L + lane], ones)
                return cc

            lax.fori_loop(0, CH_B // L, _vec, 0)

        wd = []
        for b in range(NB):
            wd.append(pltpu.async_copy(
                hist.at[pl.ds(b * L, L)],
                cnt_hbm.at[pl.ds(((g * NB + b) * NS + t) * L, L)], sem0))
        for d in wd:
            d.wait()
        return c0

    lax.fori_loop(0, NSW, _sw, 0)


@functools.partial(
    pl.kernel,
    out_type=jax.ShapeDtypeStruct((CNT + L,), jnp.int32),
    mesh=_mesh,
    compiler_params=pltpu.CompilerParams(needs_layout_passes=False),
    scratch_types=[
        pltpu.VMEM((GPT * GBLK,), jnp.int32),
        pltpu.VMEM((NS * L,), jnp.int32),
        pltpu.VMEM((L,), jnp.int32),
        pltpu.VMEM_SHARED((NS * L,), jnp.int32),
    ],
)
def _scan_kernel(cnt_hbm, offs_hbm, gbuf, gts, tail, sgt):
    core = lax.axis_index("c")
    t = lax.axis_index("s")

    @pl.when(core == 0)
    def _():
        pltpu.sync_copy(cnt_hbm.at[pl.ds(GPT * t * GBLK, GPT * GBLK)], gbuf)

        def _grp(q, carry):
            def _bkt(b, carry_b):
                def _step(k, carry_k):
                    i = q * GBLK + b * NS * L + k * L
                    x = gbuf[pl.ds(i, L)]
                    incl = plsc.cumsum(x)
                    gbuf[pl.ds(i, L)] = incl - x + carry_k
                    return carry_k + jnp.sum(x)

                ce = lax.fori_loop(0, NS, _step, carry_b)
                return jnp.bitwise_and(ce + 63, -64)

            ce = lax.fori_loop(0, NB, _bkt, carry)
            return jnp.bitwise_and(ce + 8191, -8192)

        pair_total = lax.fori_loop(0, GPT, _grp, jnp.int32(0))
        tail[pl.ds(0, L)] = jnp.full((L,), 1, jnp.int32) * pair_total
        pltpu.sync_copy(tail, sgt.at[pl.ds(t * L, L)])
        plsc.subcore_barrier()
        pltpu.sync_copy(sgt, gts)

        def _base(tp, acc2):
            v = _lane0(gts[pl.ds(tp * L, L)])
            return acc2 + jnp.where(tp < t, v, 0)

        base = lax.fori_loop(0, NS, _base, jnp.int32(0))

        def _add(i, c):
            gbuf[pl.ds(i * L, L)] = gbuf[pl.ds(i * L, L)] + base
            return c

        lax.fori_loop(0, GPT * GBLK // L, _add, 0)
        pltpu.sync_copy(gbuf, offs_hbm.at[pl.ds(GPT * t * GBLK, GPT * GBLK)])

        @pl.when(t == NS - 1)
        def _tail():
            tail[pl.ds(0, L)] = jnp.full((L,), 1, jnp.int32) * (
                base + pair_total)
            pltpu.sync_copy(tail, offs_hbm.at[pl.ds(CNT, L)])


@functools.partial(
    pl.kernel,
    out_type=[
        jax.ShapeDtypeStruct((N_PAD,), jnp.int32),
        jax.ShapeDtypeStruct((N_PAD,), jnp.float32),
    ],
    mesh=_mesh,
    compiler_params=pltpu.CompilerParams(needs_layout_passes=False),
    scratch_types=[
        pltpu.VMEM((CH_B,), jnp.int32),
        pltpu.VMEM((CH_B,), jnp.int32),
        pltpu.VMEM((CH_B,), jnp.float32),
        pltpu.VMEM((CH_B,), jnp.float32),
        pltpu.VMEM((CH_B,), jnp.int32),
        pltpu.VMEM((CH_B,), jnp.int32),
        pltpu.VMEM((NB * L,), jnp.int32),
        pltpu.VMEM((L,), jnp.int32),
        pltpu.VMEM((ZSP,), jnp.float32),
        pltpu.VMEM((ZSP,), jnp.int32),
        pltpu.VMEM_SHARED((GRP_MAX,), jnp.int32),
        pltpu.VMEM_SHARED((GRP_MAX,), jnp.float32),
        pltpu.SemaphoreType.DMA,
        pltpu.SemaphoreType.DMA,
        pltpu.SemaphoreType.DMA,
        pltpu.SemaphoreType.DMA,
        pltpu.SemaphoreType.DMA,
        pltpu.SemaphoreType.DMA,
    ],
)
def _bin_kernel(idx_hbm, val_hbm, offs_hbm, bidx_hbm, bval_hbm,
                wi0, wi1, wv0, wv1, de0, de1, own, g16, zbuf, zbuf_i,
                sp_i, sp_v, sli0, sli1, slv0, slv1, ssc0, ssc1):
    core = lax.axis_index("c")
    t = lax.axis_index("s")
    lane = lax.iota(jnp.int32, L)
    fzeros = jnp.zeros((L,), jnp.float32)
    wis = (wi0, wi1)
    wvs = (wv0, wv1)
    des = (de0, de1)
    slis = (sli0, sli1)
    slvs = (slv0, slv1)
    sscs = (ssc0, ssc1)

    izeros = jnp.zeros((L,), jnp.int32)

    def _zz(i, c):
        zbuf[pl.ds(i * L, L)] = fzeros
        zbuf_i[pl.ds(i * L, L)] = izeros
        return c

    lax.fori_loop(0, ZSP // L, _zz, 0)

    def _sw(sw, c0):
        g = core * NSW + sw
        base = core * HALF + sw * SWLEN + t * SUB
        pltpu.sync_copy(offs_hbm.at[pl.ds(g * GBLK, L)], g16)
        gb = pl.multiple_of(_lane0(g16[pl.ds(0, L)]), 2048)
        pltpu.sync_copy(offs_hbm.at[pl.ds((g + 1) * GBLK, L)], g16)
        gn = pl.multiple_of(_lane0(g16[pl.ds(0, L)]), 2048)

        zd = pltpu.async_copy(zbuf, sp_v.at[pl.ds(t * ZSP, ZSP)], ssc0)
        zdi = pltpu.async_copy(zbuf_i, sp_i.at[pl.ds(t * ZSP, ZSP)], ssc1)

        cd = []
        for b in range(NB):
            cd.append(pltpu.async_copy(
                offs_hbm.at[pl.ds(((g * NB + b) * NS + t) * L, L)],
                own.at[pl.ds(b * L, L)], sli0))
        for d in cd:
            d.wait()
        zd.wait()
        zdi.wait()
        plsc.subcore_barrier()

        ld_i = [None] * NCHB
        ld_v = [None] * NCHB
        sc_i = [None] * NCHB
        sc_v = [None] * NCHB
        ld_i[0] = pltpu.async_copy(
            idx_hbm.at[pl.ds(base, CH_B)], wis[0], slis[0])
        ld_v[0] = pltpu.async_copy(
            val_hbm.at[pl.ds(base, CH_B)], wvs[0], slvs[0])
        for ci in range(NCHB):
            b = ci % 2
            ld_i[ci].wait()
            ld_v[ci].wait()
            if ci + 1 < NCHB:
                nb = (ci + 1) % 2
                if ci >= 1:
                    sc_i[ci - 1].wait()
                    sc_v[ci - 1].wait()
                ld_i[ci + 1] = pltpu.async_copy(
                    idx_hbm.at[pl.ds(base + (ci + 1) * CH_B, CH_B)],
                    wis[nb], slis[nb])
                ld_v[ci + 1] = pltpu.async_copy(
                    val_hbm.at[pl.ds(base + (ci + 1) * CH_B, CH_B)],
                    wvs[nb], slvs[nb])

            def _vec(j, cc):
                v = wis[b][pl.ds(j * L, L)]
                addr = jnp.right_shift(v, SHIFT) * L + lane
                cur = plsc.load_gather(own, [addr])
                plsc.store_scatter(own, [addr], cur + 1)
                des[b][pl.ds(j * L, L)] = cur - gb
                return cc

            lax.fori_loop(0, CH_B // L, _vec, 0)
            sc_i[ci] = pltpu.async_copy(wis[b], sp_i.at[des[b]], sscs[b])
            sc_v[ci] = pltpu.async_copy(wvs[b], sp_v.at[des[b]], sscs[b])
        for ci in (NCHB - 2, NCHB - 1):
            sc_i[ci].wait()
            sc_v[ci].wait()
        plsc.subcore_barrier()

        nch = (gn - gb) // FCH

        def _fcond(ch):
            return ch < nch

        def _fbody(ch):
            pltpu.sync_copy(sp_i.at[pl.ds(ch * FCH, FCH)],
                            bidx_hbm.at[pl.ds(gb + ch * FCH, FCH)])
            pltpu.sync_copy(sp_v.at[pl.ds(ch * FCH, FCH)],
                            bval_hbm.at[pl.ds(gb + ch * FCH, FCH)])
            return ch + NS

        lax.while_loop(_fcond, _fbody, t)
        plsc.subcore_barrier()
        return c0

    lax.fori_loop(0, NSW, _sw, 0)


@functools.partial(
    pl.kernel,
    out_type=jax.ShapeDtypeStruct((TOTAL,), jnp.float32),
    mesh=_mesh,
    compiler_params=pltpu.CompilerParams(needs_layout_passes=False),
    scratch_types=[
        pltpu.VMEM((CHUNK,), jnp.int32),
        pltpu.VMEM((CHUNK,), jnp.float32),
        pltpu.VMEM((CHUNK,), jnp.int32),
        pltpu.VMEM((CHUNK,), jnp.float32),
        pltpu.VMEM((CHUNK,), jnp.int32),
        pltpu.VMEM((CHUNK,), jnp.float32),
        pltpu.VMEM((CHUNK,), jnp.int32),
        pltpu.VMEM((CHUNK,), jnp.float32),
        pltpu.VMEM((SEG,), jnp.float32),
        pltpu.VMEM((GPT * (NB + 1) * L,), jnp.int32),
        pltpu.VMEM_SHARED((RNG,), jnp.float32),
        pltpu.SemaphoreType.DMA,
        pltpu.SemaphoreType.DMA,
        pltpu.SemaphoreType.DMA,
        pltpu.SemaphoreType.DMA,
    ],
)
def _accum_kernel(bidx_hbm, bval_hbm, offs_hbm, out_hbm,
                  wia, wva, wib, wvb, w1i, w1v, w0i, w0v, zbuf, bnd, acc,
                  sla, slb, sca, scb):
    core = lax.axis_index("c")
    sid = lax.axis_index("s")
    fzeros = jnp.zeros((L,), jnp.float32)

    def _zz(i, c):
        zbuf[pl.ds(i * L, L)] = fzeros
        return c

    lax.fori_loop(0, SEG // L, _zz, 0)

    def _bn(b, c):
        for qq in range(GPT):
            pltpu.sync_copy(
                offs_hbm.at[pl.ds(((sid + qq * NS) * NB + b) * NS * L, L)],
                bnd.at[pl.ds((qq * (NB + 1) + b) * L, L)])
        return c

    lax.fori_loop(0, NB + 1, _bn, 0)

    def _bucket(jb, c):
        b = core * BPC + jb

        pltpu.sync_copy(zbuf, acc.at[pl.ds(sid * SEG, SEG)])
        plsc.subcore_barrier()

        lo = b * RNG

        for q in range(GPT):
            ss = _lane0(bnd[pl.ds((q * (NB + 1) + b) * L, L)])
            re = _lane0(bnd[pl.ds((q * (NB + 1) + b + 1) * L, L)])
            c0 = ss // CHUNK
            c1 = (re + CHUNK - 1) // CHUNK

            def _vecf(wi, wv, di, dv, ok):
                def _vec(j, cc):
                    lv = wi[pl.ds(j * L, L)] - lo
                    m = (lv >= 0) & (lv < RNG) & ok
                    di[pl.ds(j * L, L)] = jnp.bitwise_and(lv, RNG - 1)
                    dv[pl.ds(j * L, L)] = jnp.where(
                        m, wv[pl.ds(j * L, L)], 0.0)
                    return cc

                lax.fori_loop(0, CHUNK // L, _vec, 0)

            def _body(ci):
                cb = jnp.minimum(ci + 1, c1 - 1)
                la_i = pltpu.async_copy(bidx_hbm.at[ci], wia, sla)
                la_v = pltpu.async_copy(bval_hbm.at[ci], wva, sla)
                lb_i = pltpu.async_copy(bidx_hbm.at[cb], wib, slb)
                lb_v = pltpu.async_copy(bval_hbm.at[cb], wvb, slb)
                la_i.wait()
                la_v.wait()
                _vecf(wia, wva, w1i, w1v, ci < c1)
                da = pltpu.async_copy(w1v, acc.at[w1i], sca, add=True)
                lb_i.wait()
                lb_v.wait()
                _vecf(wib, wvb, w0i, w0v, ci + 1 < c1)
                db = pltpu.async_copy(w0v, acc.at[w0i], scb, add=True)
                da.wait()
                db.wait()
                return ci + 2

            lax.while_loop(lambda ci: ci < c1, _body, c0)

        plsc.subcore_barrier()

        pltpu.sync_copy(acc.at[pl.ds(sid * SEG, SEG)],
                        out_hbm.at[pl.ds(lo + sid * SEG, SEG)])
        plsc.subcore_barrier()
        return c

    lax.fori_loop(0, BPC, _bucket, 0)


def kernel(input, ind):
    vals = input.reshape(-1)
    idx = ind.reshape(-1).astype(jnp.int32)
    cnt = _count_kernel(idx)
    offs = _scan_kernel(cnt)
    bidx, bval = _bin_kernel(idx, vals, offs)
    out = _accum_kernel(bidx.reshape(-1, CHUNK), bval.reshape(-1, CHUNK), offs)
    return out.reshape(B_, H_ * KS, W_ * KS, C_)
